# initial kernel scaffold (unmeasured)
import jax
import jax.numpy as jnp
from jax import lax
from jax.experimental import pallas as pl
from jax.experimental.pallas import tpu as pltpu

N_DEV = 4
M = 4096
N = 2048
CHUNK = M // N_DEV

_GELU_C = 0.7978845608028654


def _gelu(y):
    return 0.5 * y * (1.0 + jnp.tanh(_GELU_C * (y + 0.044715 * y * y * y)))


def kernel(x, w_mat):
    def body(x_ref, w_ref, out_ref, comm_ref, send_sems, recv_sems):
        me = lax.axis_index("i")
        left = (me + N_DEV - 1) % N_DEV
        right = (me + 1) % N_DEV

        barrier_sem = pltpu.get_barrier_semaphore()
        for nbr in (left, right):
            pl.semaphore_signal(
                barrier_sem, inc=1,
                device_id=(nbr,), device_id_type=pl.DeviceIdType.MESH,
            )
        pl.semaphore_wait(barrier_sem, 2)

        for c in range(N_DEV):
            out_ref[pl.ds(c * CHUNK, CHUNK), :] = jnp.dot(
                x_ref[pl.ds(c * CHUNK, CHUNK), :], w_ref[:, :],
                preferred_element_type=jnp.float32,
            )

        def chunk_at(ref, idx):
            return ref.at[pl.ds(idx * CHUNK, CHUNK), :]

        for s in range(N_DEV - 1):
            send_idx = (me + N_DEV - s) % N_DEV
            recv_idx = (me + N_DEV - s - 1) % N_DEV
            rdma = pltpu.make_async_remote_copy(
                src_ref=chunk_at(out_ref, send_idx),
                dst_ref=comm_ref.at[s],
                send_sem=send_sems.at[s],
                recv_sem=recv_sems.at[s],
                device_id=(right,),
                device_id_type=pl.DeviceIdType.MESH,
            )
            rdma.start()
            rdma.wait()
            out_ref[pl.ds(recv_idx * CHUNK, CHUNK), :] += comm_ref[s]

        own = (me + 1) % N_DEV
        out_ref[pl.ds(own * CHUNK, CHUNK), :] = _gelu(
            out_ref[pl.ds(own * CHUNK, CHUNK), :]
        )

        for s in range(N_DEV - 1):
            send_idx = (me + 1 + N_DEV - s) % N_DEV
            rdma = pltpu.make_async_remote_copy(
                src_ref=chunk_at(out_ref, send_idx),
                dst_ref=chunk_at(out_ref, send_idx),
                send_sem=send_sems.at[N_DEV - 1 + s],
                recv_sem=recv_sems.at[N_DEV - 1 + s],
                device_id=(right,),
                device_id_type=pl.DeviceIdType.MESH,
            )
            rdma.start()
            rdma.wait()

    out_shape = jax.ShapeDtypeStruct((M, N), jnp.float32)
    return pl.pallas_call(
        body,
        out_shape=out_shape,
        in_specs=[
            pl.BlockSpec(memory_space=pltpu.VMEM),
            pl.BlockSpec(memory_space=pltpu.VMEM),
        ],
        out_specs=pl.BlockSpec(memory_space=pltpu.VMEM),
        scratch_shapes=[
            pltpu.VMEM((N_DEV - 1, CHUNK, N), jnp.float32),
            pltpu.SemaphoreType.DMA((2 * (N_DEV - 1),)),
            pltpu.SemaphoreType.DMA((2 * (N_DEV - 1),)),
        ],
        compiler_params=pltpu.CompilerParams(collective_id=0),
    )(x, w_mat)


# baseline (device time: 617747 ns/iter reference)
import jax
import jax.numpy as jnp
from jax import lax
from jax.experimental import pallas as pl
from jax.experimental.pallas import tpu as pltpu

N_DEV = 4
M = 4096
K_SHARD = 1024
N = 2048
CHUNK = M // N_DEV

_GELU_C = 0.7978845608028654


def _gelu(y):
    return 0.5 * y * (1.0 + jnp.tanh(_GELU_C * (y + 0.044715 * y * y * y)))


def kernel(x, w_mat):
    def body(x_hbm, w_ref, out_ref, xbuf, comm_ref, copy_sems, send_sems,
             recv_sems, credit_sem):
        me = lax.axis_index("i")
        left = (me + N_DEV - 1) % N_DEV
        right = (me + 1) % N_DEV

        barrier_sem = pltpu.get_barrier_semaphore()
        for nbr in (left, right):
            pl.semaphore_signal(
                barrier_sem, inc=1,
                device_id=(nbr,), device_id_type=pl.DeviceIdType.MESH,
            )
        pl.semaphore_wait(barrier_sem, 2)

        def x_copy(c, slot):
            return pltpu.make_async_copy(
                x_hbm.at[pl.ds(c * CHUNK, CHUNK), :],
                xbuf.at[slot],
                copy_sems.at[slot],
            )

        x_copy(0, 0).start()
        for c in range(N_DEV):
            if c + 1 < N_DEV:
                x_copy(c + 1, (c + 1) % 2).start()
            x_copy(c, c % 2).wait()
            out_ref[pl.ds(c * CHUNK, CHUNK), :] = jnp.dot(
                xbuf[c % 2], w_ref[:, :],
                preferred_element_type=jnp.float32,
            )

        def chunk_at(ref, idx):
            return ref.at[pl.ds(idx * CHUNK, CHUNK), :]

        for s in range(N_DEV - 1):
            send_idx = (me + N_DEV - s) % N_DEV
            recv_idx = (me + N_DEV - s - 1) % N_DEV
            if s > 0:
                pl.semaphore_wait(credit_sem, 1)
            rdma = pltpu.make_async_remote_copy(
                src_ref=chunk_at(out_ref, send_idx),
                dst_ref=comm_ref,
                send_sem=send_sems.at[0],
                recv_sem=recv_sems.at[0],
                device_id=(right,),
                device_id_type=pl.DeviceIdType.MESH,
            )
            rdma.start()
            rdma.wait()
            out_ref[pl.ds(recv_idx * CHUNK, CHUNK), :] += comm_ref[:, :]
            if s + 1 < N_DEV - 1:
                pl.semaphore_signal(
                    credit_sem, inc=1,
                    device_id=(left,), device_id_type=pl.DeviceIdType.MESH,
                )

        own = (me + 1) % N_DEV
        out_ref[pl.ds(own * CHUNK, CHUNK), :] = _gelu(
            out_ref[pl.ds(own * CHUNK, CHUNK), :]
        )

        for s in range(N_DEV - 1):
            send_idx = (me + 1 + N_DEV - s) % N_DEV
            rdma = pltpu.make_async_remote_copy(
                src_ref=chunk_at(out_ref, send_idx),
                dst_ref=chunk_at(out_ref, send_idx),
                send_sem=send_sems.at[1 + s],
                recv_sem=recv_sems.at[1 + s],
                device_id=(right,),
                device_id_type=pl.DeviceIdType.MESH,
            )
            rdma.start()
            rdma.wait()

    out_shape = jax.ShapeDtypeStruct((M, N), jnp.float32)
    return pl.pallas_call(
        body,
        out_shape=out_shape,
        in_specs=[
            pl.BlockSpec(memory_space=pltpu.HBM),
            pl.BlockSpec(memory_space=pltpu.VMEM),
        ],
        out_specs=pl.BlockSpec(memory_space=pltpu.VMEM),
        scratch_shapes=[
            pltpu.VMEM((2, CHUNK, K_SHARD), jnp.float32),
            pltpu.VMEM((CHUNK, N), jnp.float32),
            pltpu.SemaphoreType.DMA((2,)),
            pltpu.SemaphoreType.DMA((4,)),
            pltpu.SemaphoreType.DMA((4,)),
            pltpu.SemaphoreType.REGULAR,
        ],
        compiler_params=pltpu.CompilerParams(
            collective_id=0,
            vmem_limit_bytes=100 * 1024 * 1024,
        ),
    )(x, w_mat)


# device time: 350507 ns/iter; 1.7624x vs baseline; 1.7624x over previous
import jax
import jax.numpy as jnp
from jax import lax
from jax.experimental import pallas as pl
from jax.experimental.pallas import tpu as pltpu

N_DEV = 4
M = 4096
K_SHARD = 1024
N = 2048
CHUNK = M // N_DEV
SLAB = 512
HALF = N // 2

_GELU_C = 0.7978845608028654


def _gelu(y):
    return 0.5 * y * (1.0 + jnp.tanh(_GELU_C * (y + 0.044715 * y * y * y)))


def kernel(x, w_mat):
    def body(x_hbm, w_ref, out_ref, xbuf, comm_ref, copy_sems, send_sems,
             recv_sems, credit_sems):
        me = lax.axis_index("i")
        left = (me + N_DEV - 1) % N_DEV
        right = (me + 1) % N_DEV

        barrier_sem = pltpu.get_barrier_semaphore()
        for nbr in (left, right):
            pl.semaphore_signal(
                barrier_sem, inc=1,
                device_id=(nbr,), device_id_type=pl.DeviceIdType.MESH,
            )
        pl.semaphore_wait(barrier_sem, 2)

        n_slabs = M // SLAB
        def x_copy(c, slot):
            return pltpu.make_async_copy(
                x_hbm.at[pl.ds(c * SLAB, SLAB), :],
                xbuf.at[slot],
                copy_sems.at[slot],
            )

        x_copy(0, 0).start()
        for c in range(n_slabs):
            if c + 1 < n_slabs:
                x_copy(c + 1, (c + 1) % 2).start()
            x_copy(c, c % 2).wait()
            out_ref[pl.ds(c * SLAB, SLAB), :] = jnp.dot(
                xbuf[c % 2], w_ref[:, :],
                preferred_element_type=jnp.float32,
            )

        def half_at(ref, idx, d):
            return ref.at[pl.ds(idx * CHUNK, CHUNK), pl.ds(d * HALF, HALF)]

        dests = (right, left)
        upstreams = (left, right)

        for s in range(N_DEV - 1):
            send_idx = [(me + N_DEV - s) % N_DEV, (me + s) % N_DEV]
            recv_idx = [(me + N_DEV - s - 1) % N_DEV, (me + s + 1) % N_DEV]
            rdmas = []
            for d in range(2):
                if s > 0:
                    pl.semaphore_wait(credit_sems.at[d], 1)
                rdma = pltpu.make_async_remote_copy(
                    src_ref=half_at(out_ref, send_idx[d], d),
                    dst_ref=comm_ref.at[d],
                    send_sem=send_sems.at[d, 0],
                    recv_sem=recv_sems.at[d, 0],
                    device_id=(dests[d],),
                    device_id_type=pl.DeviceIdType.MESH,
                )
                rdma.start()
                rdmas.append(rdma)
            for d in range(2):
                rdmas[d].wait()
                out_ref[pl.ds(recv_idx[d] * CHUNK, CHUNK),
                        pl.ds(d * HALF, HALF)] += comm_ref[d]
                if s + 1 < N_DEV - 1:
                    pl.semaphore_signal(
                        credit_sems.at[d], inc=1,
                        device_id=(upstreams[d],),
                        device_id_type=pl.DeviceIdType.MESH,
                    )

        own = [(me + 1) % N_DEV, (me + N_DEV - 1) % N_DEV]
        for d in range(2):
            rows = pl.ds(own[d] * CHUNK, CHUNK)
            cols = pl.ds(d * HALF, HALF)
            out_ref[rows, cols] = _gelu(out_ref[rows, cols])

        for s in range(N_DEV - 1):
            send_idx = [(me + 1 + N_DEV - s) % N_DEV,
                        (me + N_DEV - 1 + s) % N_DEV]
            rdmas = []
            for d in range(2):
                rdma = pltpu.make_async_remote_copy(
                    src_ref=half_at(out_ref, send_idx[d], d),
                    dst_ref=half_at(out_ref, send_idx[d], d),
                    send_sem=send_sems.at[d, 1 + s],
                    recv_sem=recv_sems.at[d, 1 + s],
                    device_id=(dests[d],),
                    device_id_type=pl.DeviceIdType.MESH,
                )
                rdma.start()
                rdmas.append(rdma)
            for d in range(2):
                rdmas[d].wait()

    out_shape = jax.ShapeDtypeStruct((M, N), jnp.float32)
    return pl.pallas_call(
        body,
        out_shape=out_shape,
        in_specs=[
            pl.BlockSpec(memory_space=pltpu.HBM),
            pl.BlockSpec(memory_space=pltpu.VMEM),
        ],
        out_specs=pl.BlockSpec(memory_space=pltpu.VMEM),
        scratch_shapes=[
            pltpu.VMEM((2, SLAB, K_SHARD), jnp.float32),
            pltpu.VMEM((2, CHUNK, HALF), jnp.float32),
            pltpu.SemaphoreType.DMA((2,)),
            pltpu.SemaphoreType.DMA((2, 4)),
            pltpu.SemaphoreType.DMA((2, 4)),
            pltpu.SemaphoreType.REGULAR((2,)),
        ],
        compiler_params=pltpu.CompilerParams(
            collective_id=0,
            vmem_limit_bytes=100 * 1024 * 1024,
        ),
    )(x, w_mat)


# device time: 319388 ns/iter; 1.9342x vs baseline; 1.0974x over previous
import jax
import jax.numpy as jnp
from jax import lax
from jax.experimental import pallas as pl
from jax.experimental.pallas import tpu as pltpu

N_DEV = 4
M = 4096
K_SHARD = 1024
N = 2048
CHUNK = M // N_DEV
SLAB = 512
HALF = N // 2
SUBS = 4
SUBR = CHUNK // SUBS
N_HOP = N_DEV - 1

_GELU_C = 0.7978845608028654


def _gelu(y):
    return 0.5 * y * (1.0 + jnp.tanh(_GELU_C * (y + 0.044715 * y * y * y)))


def kernel(x, w_mat):
    def body(x_hbm, w_ref, out_ref, xbuf, comm_ref, copy_sems,
             rs_send_sems, rs_recv_sems, ag_send_sems, ag_recv_sems,
             credit_sems):
        me = lax.axis_index("i")
        left = (me + N_DEV - 1) % N_DEV
        right = (me + 1) % N_DEV

        barrier_sem = pltpu.get_barrier_semaphore()
        for nbr in (left, right):
            pl.semaphore_signal(
                barrier_sem, inc=1,
                device_id=(nbr,), device_id_type=pl.DeviceIdType.MESH,
            )
        pl.semaphore_wait(barrier_sem, 2)

        dests = (right, left)
        upstreams = (left, right)

        def sub_at(chunk_idx, d, k):
            return out_ref.at[pl.ds(chunk_idx * CHUNK + k * SUBR, SUBR),
                              pl.ds(d * HALF, HALF)]

        def comm_at(d, k):
            return comm_ref.at[d, pl.ds(k * SUBR, SUBR), :]

        rs_send_idx = [[(me + N_DEV - s) % N_DEV for s in range(N_HOP)],
                       [(me + s) % N_DEV for s in range(N_HOP)]]
        rs_recv_idx = [[(me + N_DEV - s - 1) % N_DEV for s in range(N_HOP)],
                       [(me + s + 1) % N_DEV for s in range(N_HOP)]]
        ag_send_idx = [[(me + 1 + N_DEV - h) % N_DEV for h in range(N_HOP)],
                       [(me + N_DEV - 1 + h) % N_DEV for h in range(N_HOP)]]
        ag_recv_idx = [[(me + N_DEV - h) % N_DEV for h in range(N_HOP)],
                       [(me + h) % N_DEV for h in range(N_HOP)]]

        def rs_rdma(s, d, k):
            return pltpu.make_async_remote_copy(
                src_ref=sub_at(rs_send_idx[d][s], d, k),
                dst_ref=comm_at(d, k),
                send_sem=rs_send_sems.at[d, k],
                recv_sem=rs_recv_sems.at[d, k],
                device_id=(dests[d],),
                device_id_type=pl.DeviceIdType.MESH,
            )

        def ag_rdma(h, d, k, for_recv=False):
            idx = ag_recv_idx[d][h] if for_recv else ag_send_idx[d][h]
            return pltpu.make_async_remote_copy(
                src_ref=sub_at(idx, d, k),
                dst_ref=sub_at(idx, d, k),
                send_sem=ag_send_sems.at[d, h, k],
                recv_sem=ag_recv_sems.at[d, h, k],
                device_id=(dests[d],),
                device_id_type=pl.DeviceIdType.MESH,
            )

        n_slabs = M // SLAB
        slabs_per_chunk = CHUNK // SLAB
        chunk_order = [me, (me + N_DEV - 1) % N_DEV, (me + 1) % N_DEV,
                       (me + 2) % N_DEV]
        slab_rows = [chunk_order[j // slabs_per_chunk] * CHUNK
                     + (j % slabs_per_chunk) * SLAB for j in range(n_slabs)]

        def x_copy(j, slot):
            return pltpu.make_async_copy(
                x_hbm.at[pl.ds(slab_rows[j], SLAB), :],
                xbuf.at[slot],
                copy_sems.at[slot],
            )

        def gemm_slab(j):
            if j == 0:
                x_copy(0, 0).start()
            if j + 1 < n_slabs:
                x_copy(j + 1, (j + 1) % 2).start()
            x_copy(j, j % 2).wait()
            out_ref[pl.ds(slab_rows[j], SLAB), :] = jnp.dot(
                xbuf[j % 2], w_ref[:, :],
                preferred_element_type=jnp.float32,
            )

        for j in range(slabs_per_chunk):
            gemm_slab(j)

        prev_rs = {}
        for k in range(SUBS):
            for d in range(2):
                r = rs_rdma(0, d, k)
                r.start()
                prev_rs[(d, k)] = r

        for j in range(slabs_per_chunk, n_slabs):
            gemm_slab(j)

        ag_started = {}
        for s in range(N_HOP):
            for k in range(SUBS):
                for d in range(2):
                    rs_rdma(s, d, k).wait_recv()
                    rows = pl.ds(rs_recv_idx[d][s] * CHUNK + k * SUBR, SUBR)
                    cols = pl.ds(d * HALF, HALF)
                    out_ref[rows, cols] += comm_ref[d, pl.ds(k * SUBR, SUBR), :]
                    if s + 1 < N_HOP:
                        pl.semaphore_signal(
                            credit_sems.at[d, k], inc=1,
                            device_id=(upstreams[d],),
                            device_id_type=pl.DeviceIdType.MESH,
                        )
                        prev_rs[(d, k)].wait_send()
                        pl.semaphore_wait(credit_sems.at[d, k], 1)
                        r = rs_rdma(s + 1, d, k)
                        r.start()
                        prev_rs[(d, k)] = r
                    else:
                        out_ref[rows, cols] = _gelu(out_ref[rows, cols])
                        r = ag_rdma(0, d, k)
                        r.start()
                        ag_started[(0, d, k)] = r

        for h in range(N_HOP):
            for k in range(SUBS):
                for d in range(2):
                    ag_rdma(h, d, k, for_recv=True).wait_recv()
                    if h + 1 < N_HOP:
                        r = ag_rdma(h + 1, d, k)
                        r.start()
                        ag_started[(h + 1, d, k)] = r

        for r in prev_rs.values():
            r.wait_send()
        for r in ag_started.values():
            r.wait_send()

    out_shape = jax.ShapeDtypeStruct((M, N), jnp.float32)
    return pl.pallas_call(
        body,
        out_shape=out_shape,
        in_specs=[
            pl.BlockSpec(memory_space=pltpu.HBM),
            pl.BlockSpec(memory_space=pltpu.VMEM),
        ],
        out_specs=pl.BlockSpec(memory_space=pltpu.VMEM),
        scratch_shapes=[
            pltpu.VMEM((2, SLAB, K_SHARD), jnp.float32),
            pltpu.VMEM((2, CHUNK, HALF), jnp.float32),
            pltpu.SemaphoreType.DMA((2,)),
            pltpu.SemaphoreType.DMA((2, SUBS)),
            pltpu.SemaphoreType.DMA((2, SUBS)),
            pltpu.SemaphoreType.DMA((2, N_HOP, SUBS)),
            pltpu.SemaphoreType.DMA((2, N_HOP, SUBS)),
            pltpu.SemaphoreType.REGULAR((2, SUBS)),
        ],
        compiler_params=pltpu.CompilerParams(
            collective_id=0,
            vmem_limit_bytes=100 * 1024 * 1024,
        ),
    )(x, w_mat)


# device time: 316116 ns/iter; 1.9542x vs baseline; 1.0104x over previous
import jax
import jax.numpy as jnp
from jax import lax
from jax.experimental import pallas as pl
from jax.experimental.pallas import tpu as pltpu

N_DEV = 4
M = 4096
K_SHARD = 1024
N = 2048
CHUNK = M // N_DEV
SLAB = 256
HALF = N // 2
SUBS = 8
SUBR = CHUNK // SUBS
N_HOP = N_DEV - 1

_GELU_C = 0.7978845608028654


def _gelu(y):
    return 0.5 * y * (1.0 + jnp.tanh(_GELU_C * (y + 0.044715 * y * y * y)))


def kernel(x, w_mat):
    def body(x_hbm, w_ref, out_ref, xbuf, comm_ref, copy_sems,
             rs_send_sems, rs_recv_sems, ag_send_sems, ag_recv_sems,
             credit_sems):
        me = lax.axis_index("i")
        left = (me + N_DEV - 1) % N_DEV
        right = (me + 1) % N_DEV

        barrier_sem = pltpu.get_barrier_semaphore()
        for nbr in (left, right):
            pl.semaphore_signal(
                barrier_sem, inc=1,
                device_id=(nbr,), device_id_type=pl.DeviceIdType.MESH,
            )
        pl.semaphore_wait(barrier_sem, 2)

        dests = (right, left)
        upstreams = (left, right)

        def sub_at(chunk_idx, d, k):
            return out_ref.at[pl.ds(chunk_idx * CHUNK + k * SUBR, SUBR),
                              pl.ds(d * HALF, HALF)]

        def comm_at(d, k):
            return comm_ref.at[d, pl.ds(k * SUBR, SUBR), :]

        rs_send_idx = [[(me + N_DEV - s) % N_DEV for s in range(N_HOP)],
                       [(me + s) % N_DEV for s in range(N_HOP)]]
        rs_recv_idx = [[(me + N_DEV - s - 1) % N_DEV for s in range(N_HOP)],
                       [(me + s + 1) % N_DEV for s in range(N_HOP)]]
        ag_send_idx = [[(me + 1 + N_DEV - h) % N_DEV for h in range(N_HOP)],
                       [(me + N_DEV - 1 + h) % N_DEV for h in range(N_HOP)]]
        ag_recv_idx = [[(me + N_DEV - h) % N_DEV for h in range(N_HOP)],
                       [(me + h) % N_DEV for h in range(N_HOP)]]

        def rs_rdma(s, d, k):
            return pltpu.make_async_remote_copy(
                src_ref=sub_at(rs_send_idx[d][s], d, k),
                dst_ref=comm_at(d, k),
                send_sem=rs_send_sems.at[d, k],
                recv_sem=rs_recv_sems.at[d, k],
                device_id=(dests[d],),
                device_id_type=pl.DeviceIdType.MESH,
            )

        def ag_rdma(h, d, k, for_recv=False):
            idx = ag_recv_idx[d][h] if for_recv else ag_send_idx[d][h]
            return pltpu.make_async_remote_copy(
                src_ref=sub_at(idx, d, k),
                dst_ref=sub_at(idx, d, k),
                send_sem=ag_send_sems.at[d, h, k],
                recv_sem=ag_recv_sems.at[d, h, k],
                device_id=(dests[d],),
                device_id_type=pl.DeviceIdType.MESH,
            )

        n_slabs = M // SLAB
        slabs_per_chunk = CHUNK // SLAB
        chunk_order = [me, (me + N_DEV - 1) % N_DEV, (me + 1) % N_DEV,
                       (me + 2) % N_DEV]
        slab_rows = [chunk_order[j // slabs_per_chunk] * CHUNK
                     + (j % slabs_per_chunk) * SLAB for j in range(n_slabs)]

        def x_copy(j, slot):
            return pltpu.make_async_copy(
                x_hbm.at[pl.ds(slab_rows[j], SLAB), :],
                xbuf.at[slot],
                copy_sems.at[slot],
            )

        def gemm_slab(j):
            if j == 0:
                x_copy(0, 0).start()
            if j + 1 < n_slabs:
                x_copy(j + 1, (j + 1) % 2).start()
            x_copy(j, j % 2).wait()
            out_ref[pl.ds(slab_rows[j], SLAB), :] = jnp.dot(
                xbuf[j % 2], w_ref[:, :],
                preferred_element_type=jnp.float32,
            )

        prev_rs = {}
        subs_per_slab = SLAB // SUBR
        for j in range(slabs_per_chunk):
            gemm_slab(j)
            for k in range(j * subs_per_slab, (j + 1) * subs_per_slab):
                for d in range(2):
                    r = rs_rdma(0, d, k)
                    r.start()
                    prev_rs[(d, k)] = r

        for j in range(slabs_per_chunk, n_slabs):
            gemm_slab(j)

        ag_started = {}
        for s in range(N_HOP):
            for k in range(SUBS):
                for d in range(2):
                    rs_rdma(s, d, k).wait_recv()
                    rows = pl.ds(rs_recv_idx[d][s] * CHUNK + k * SUBR, SUBR)
                    cols = pl.ds(d * HALF, HALF)
                    out_ref[rows, cols] += comm_ref[d, pl.ds(k * SUBR, SUBR), :]
                    if s + 1 < N_HOP:
                        pl.semaphore_signal(
                            credit_sems.at[d, k], inc=1,
                            device_id=(upstreams[d],),
                            device_id_type=pl.DeviceIdType.MESH,
                        )
                        prev_rs[(d, k)].wait_send()
                        pl.semaphore_wait(credit_sems.at[d, k], 1)
                        r = rs_rdma(s + 1, d, k)
                        r.start()
                        prev_rs[(d, k)] = r
                    else:
                        out_ref[rows, cols] = _gelu(out_ref[rows, cols])
                        r = ag_rdma(0, d, k)
                        r.start()
                        ag_started[(0, d, k)] = r

        for h in range(N_HOP):
            for k in range(SUBS):
                for d in range(2):
                    ag_rdma(h, d, k, for_recv=True).wait_recv()
                    if h + 1 < N_HOP:
                        r = ag_rdma(h + 1, d, k)
                        r.start()
                        ag_started[(h + 1, d, k)] = r

        for r in prev_rs.values():
            r.wait_send()
        for r in ag_started.values():
            r.wait_send()

    out_shape = jax.ShapeDtypeStruct((M, N), jnp.float32)
    return pl.pallas_call(
        body,
        out_shape=out_shape,
        in_specs=[
            pl.BlockSpec(memory_space=pltpu.HBM),
            pl.BlockSpec(memory_space=pltpu.VMEM),
        ],
        out_specs=pl.BlockSpec(memory_space=pltpu.VMEM),
        scratch_shapes=[
            pltpu.VMEM((2, SLAB, K_SHARD), jnp.float32),
            pltpu.VMEM((2, CHUNK, HALF), jnp.float32),
            pltpu.SemaphoreType.DMA((2,)),
            pltpu.SemaphoreType.DMA((2, SUBS)),
            pltpu.SemaphoreType.DMA((2, SUBS)),
            pltpu.SemaphoreType.DMA((2, N_HOP, SUBS)),
            pltpu.SemaphoreType.DMA((2, N_HOP, SUBS)),
            pltpu.SemaphoreType.REGULAR((2, SUBS)),
        ],
        compiler_params=pltpu.CompilerParams(
            collective_id=0,
            vmem_limit_bytes=100 * 1024 * 1024,
        ),
    )(x, w_mat)


# device time: 308398 ns/iter; 2.0031x vs baseline; 1.0250x over previous
import jax
import jax.numpy as jnp
from jax import lax
from jax.experimental import pallas as pl
from jax.experimental.pallas import tpu as pltpu

N_DEV = 4
M = 4096
K_SHARD = 1024
N = 2048
CHUNK = M // N_DEV
SLAB = 256
HALF = N // 2
SUBS = 8
SUBR = CHUNK // SUBS
N_HOP = N_DEV - 1

_GELU_C = 0.7978845608028654


def _gelu(y):
    return 0.5 * y * (1.0 + jnp.tanh(_GELU_C * (y + 0.044715 * y * y * y)))


def kernel(x, w_mat):
    def body(x_hbm, w_ref, out_hbm, xbuf, comm_ref, copy_sems,
             rs_send_sems, rs_recv_sems, ag_send_sems, ag_recv_sems,
             credit_sems, acc_ref, flush_sems):
        me = lax.axis_index("i")
        out_ref = acc_ref
        left = (me + N_DEV - 1) % N_DEV
        right = (me + 1) % N_DEV

        barrier_sem = pltpu.get_barrier_semaphore()
        for nbr in (left, right):
            pl.semaphore_signal(
                barrier_sem, inc=1,
                device_id=(nbr,), device_id_type=pl.DeviceIdType.MESH,
            )
        pl.semaphore_wait(barrier_sem, 2)

        dests = (right, left)
        upstreams = (left, right)

        def sub_at(chunk_idx, d, k):
            return out_ref.at[pl.ds(chunk_idx * CHUNK + k * SUBR, SUBR),
                              pl.ds(d * HALF, HALF)]

        def comm_at(d, k):
            return comm_ref.at[d, pl.ds(k * SUBR, SUBR), :]

        rs_send_idx = [[(me + N_DEV - s) % N_DEV for s in range(N_HOP)],
                       [(me + s) % N_DEV for s in range(N_HOP)]]
        rs_recv_idx = [[(me + N_DEV - s - 1) % N_DEV for s in range(N_HOP)],
                       [(me + s + 1) % N_DEV for s in range(N_HOP)]]
        ag_send_idx = [[(me + 1 + N_DEV - h) % N_DEV for h in range(N_HOP)],
                       [(me + N_DEV - 1 + h) % N_DEV for h in range(N_HOP)]]
        ag_recv_idx = [[(me + N_DEV - h) % N_DEV for h in range(N_HOP)],
                       [(me + h) % N_DEV for h in range(N_HOP)]]

        def rs_rdma(s, d, k):
            return pltpu.make_async_remote_copy(
                src_ref=sub_at(rs_send_idx[d][s], d, k),
                dst_ref=comm_at(d, k),
                send_sem=rs_send_sems.at[d, k],
                recv_sem=rs_recv_sems.at[d, k],
                device_id=(dests[d],),
                device_id_type=pl.DeviceIdType.MESH,
            )

        def ag_rdma(h, d, k, for_recv=False):
            idx = ag_recv_idx[d][h] if for_recv else ag_send_idx[d][h]
            return pltpu.make_async_remote_copy(
                src_ref=sub_at(idx, d, k),
                dst_ref=sub_at(idx, d, k),
                send_sem=ag_send_sems.at[d, h, k],
                recv_sem=ag_recv_sems.at[d, h, k],
                device_id=(dests[d],),
                device_id_type=pl.DeviceIdType.MESH,
            )

        n_slabs = M // SLAB
        slabs_per_chunk = CHUNK // SLAB
        chunk_order = [me, (me + N_DEV - 1) % N_DEV, (me + 1) % N_DEV,
                       (me + 2) % N_DEV]
        slab_rows = [chunk_order[j // slabs_per_chunk] * CHUNK
                     + (j % slabs_per_chunk) * SLAB for j in range(n_slabs)]

        def x_copy(j, slot):
            return pltpu.make_async_copy(
                x_hbm.at[pl.ds(slab_rows[j], SLAB), :],
                xbuf.at[slot],
                copy_sems.at[slot],
            )

        def gemm_slab(j):
            if j == 0:
                x_copy(0, 0).start()
            if j + 1 < n_slabs:
                x_copy(j + 1, (j + 1) % 2).start()
            x_copy(j, j % 2).wait()
            out_ref[pl.ds(slab_rows[j], SLAB), :] = jnp.dot(
                xbuf[j % 2], w_ref[:, :],
                preferred_element_type=jnp.float32,
            )

        prev_rs = {}
        subs_per_slab = SLAB // SUBR
        for j in range(slabs_per_chunk):
            gemm_slab(j)
            for k in range(j * subs_per_slab, (j + 1) * subs_per_slab):
                for d in range(2):
                    r = rs_rdma(0, d, k)
                    r.start()
                    prev_rs[(d, k)] = r

        for j in range(slabs_per_chunk, n_slabs):
            gemm_slab(j)

        def chunk_flush(idx, d, sem):
            rows = pl.ds(idx * CHUNK, CHUNK)
            cols = pl.ds(d * HALF, HALF)
            return pltpu.make_async_copy(
                acc_ref.at[rows, cols], out_hbm.at[rows, cols], sem)

        ag_started = {}
        flushes = []
        for s in range(N_HOP):
            for k in range(SUBS):
                for d in range(2):
                    rs_rdma(s, d, k).wait_recv()
                    rows = pl.ds(rs_recv_idx[d][s] * CHUNK + k * SUBR, SUBR)
                    cols = pl.ds(d * HALF, HALF)
                    out_ref[rows, cols] += comm_ref[d, pl.ds(k * SUBR, SUBR), :]
                    if s + 1 < N_HOP:
                        pl.semaphore_signal(
                            credit_sems.at[d, k], inc=1,
                            device_id=(upstreams[d],),
                            device_id_type=pl.DeviceIdType.MESH,
                        )
                        prev_rs[(d, k)].wait_send()
                        pl.semaphore_wait(credit_sems.at[d, k], 1)
                        r = rs_rdma(s + 1, d, k)
                        r.start()
                        prev_rs[(d, k)] = r
                    else:
                        out_ref[rows, cols] = _gelu(out_ref[rows, cols])
                        r = ag_rdma(0, d, k)
                        r.start()
                        ag_started[(0, d, k)] = r
                        if k == SUBS - 1:
                            f = chunk_flush(rs_recv_idx[d][s], d,
                                            flush_sems.at[d, 0])
                            f.start()
                            flushes.append(f)

        for h in range(N_HOP):
            for k in range(SUBS):
                for d in range(2):
                    ag_rdma(h, d, k, for_recv=True).wait_recv()
                    if h + 1 < N_HOP:
                        r = ag_rdma(h + 1, d, k)
                        r.start()
                        ag_started[(h + 1, d, k)] = r
                    if k == SUBS - 1:
                        f = chunk_flush(ag_recv_idx[d][h], d,
                                        flush_sems.at[d, 1 + h])
                        f.start()
                        flushes.append(f)

        for r in prev_rs.values():
            r.wait_send()
        for r in ag_started.values():
            r.wait_send()
        for f in flushes:
            f.wait()

    out_shape = jax.ShapeDtypeStruct((M, N), jnp.float32)
    return pl.pallas_call(
        body,
        out_shape=out_shape,
        in_specs=[
            pl.BlockSpec(memory_space=pltpu.HBM),
            pl.BlockSpec(memory_space=pltpu.VMEM),
        ],
        out_specs=pl.BlockSpec(memory_space=pltpu.HBM),
        scratch_shapes=[
            pltpu.VMEM((2, SLAB, K_SHARD), jnp.float32),
            pltpu.VMEM((2, CHUNK, HALF), jnp.float32),
            pltpu.SemaphoreType.DMA((2,)),
            pltpu.SemaphoreType.DMA((2, SUBS)),
            pltpu.SemaphoreType.DMA((2, SUBS)),
            pltpu.SemaphoreType.DMA((2, N_HOP, SUBS)),
            pltpu.SemaphoreType.DMA((2, N_HOP, SUBS)),
            pltpu.SemaphoreType.REGULAR((2, SUBS)),
            pltpu.VMEM((M, N), jnp.float32),
            pltpu.SemaphoreType.DMA((2, N_DEV)),
        ],
        compiler_params=pltpu.CompilerParams(
            collective_id=0,
            vmem_limit_bytes=100 * 1024 * 1024,
        ),
    )(x, w_mat)
